# Initial kernel scaffold; baseline (speedup 1.0000x reference)
import functools

import jax
import jax.numpy as jnp
from jax import lax
from jax.experimental import pallas as pl
from jax.experimental.pallas import tpu as pltpu
from jax.experimental.pallas import tpu_sc as plsc

B, V, D, L = 1024, 1000, 32, 200
NW = 32
ROWS_PER_W = B * L // NW
BATCH_PER_W = B // NW
IDX_W = 128
IDX_ROWS = ROWS_PER_W // IDX_W
CHUNK = 1280
N_CHUNK = ROWS_PER_W // CHUNK
PIECES = CHUNK // IDX_W
DIV_M, DIV_S = 5243, 20


def _body(tab_hbm, inp_hbm, out_hbm, idx_raw, gid, rows, sem):
    c = lax.axis_index("c")
    s = lax.axis_index("s")
    wid = s * 2 + c
    r0 = wid * ROWS_PER_W

    pltpu.sync_copy(inp_hbm.at[pl.ds(wid * IDX_ROWS, IDX_ROWS)], idx_raw)

    wbase = wid * BATCH_PER_W * V
    lanes = lax.iota(jnp.int32, 16)

    def vec_body(i, carry):
        row = i >> 3
        col = (i & 7) * 16
        pos = i * 16 + lanes
        b_local = lax.shift_right_logical(pos * DIV_M, DIV_S)
        raw = idx_raw[row, pl.ds(col, 16)]
        gid[row, pl.ds(col, 16)] = raw + (wbase + b_local * V)
        return carry

    lax.fori_loop(0, IDX_ROWS * 8, vec_body, 0)

    for ch in range(N_CHUNK):
        waits = []
        for p in range(PIECES):
            waits.append(pltpu.async_copy(
                tab_hbm.at[gid.at[ch * PIECES + p]],
                rows.at[pl.ds(p * IDX_W, IDX_W)], sem))
        for w in waits:
            w.wait()
        pltpu.sync_copy(rows, out_hbm.at[pl.ds(r0 + ch * CHUNK, CHUNK)])


@jax.jit
def _run(tab, inp2):
    mesh = plsc.VectorSubcoreMesh(core_axis_name="c", subcore_axis_name="s")
    f = pl.kernel(
        _body,
        out_type=jax.ShapeDtypeStruct((B * L, D), jnp.float32),
        mesh=mesh,
        scratch_types=[
            pltpu.VMEM((IDX_ROWS, IDX_W), jnp.int32),
            pltpu.VMEM((IDX_ROWS, IDX_W), jnp.int32),
            pltpu.VMEM((CHUNK, D), jnp.float32),
            pltpu.SemaphoreType.DMA,
        ],
    )
    return f(tab, inp2)


def kernel(input, weight):
    tab = weight.reshape(B * V, D)
    inp2 = input.reshape(NW * IDX_ROWS, IDX_W).astype(jnp.int32)
    out = _run(tab, inp2)
    return out.reshape(B, L, D)


# speedup vs baseline: 1.1282x; 1.1282x over previous
import functools

import jax
import jax.numpy as jnp
from jax import lax
from jax.experimental import pallas as pl
from jax.experimental.pallas import tpu as pltpu
from jax.experimental.pallas import tpu_sc as plsc

B, V, D, L = 1024, 1000, 32, 200
NW = 32
ROWS_PER_W = B * L // NW
BATCH_PER_W = B // NW
IDX_W = 128
IDX_ROWS = ROWS_PER_W // IDX_W
CHUNK = 1280
N_CHUNK = ROWS_PER_W // CHUNK
PIECES = CHUNK // IDX_W
DIV_M, DIV_S = 5243, 20


def _body(tab_hbm, inp_hbm, out_hbm, idx_raw, gid, rows, sem):
    c = lax.axis_index("c")
    s = lax.axis_index("s")
    wid = s * 2 + c
    r0 = wid * ROWS_PER_W

    pltpu.sync_copy(inp_hbm.at[pl.ds(r0, ROWS_PER_W)], idx_raw)

    wbase = wid * BATCH_PER_W * V
    lanes = lax.iota(jnp.int32, 16)

    def vec_body(i, carry):
        row = i >> 3
        col = (i & 7) * 16
        pos = i * 16 + lanes
        b_local = lax.shift_right_logical(pos * DIV_M, DIV_S)
        raw = idx_raw[pl.ds(i * 16, 16)]
        gid[row, pl.ds(col, 16)] = raw + (wbase + b_local * V)
        return carry

    lax.fori_loop(0, IDX_ROWS * 8, vec_body, 0)

    for ch in range(N_CHUNK):
        waits = []
        for p in range(PIECES):
            waits.append(pltpu.async_copy(
                tab_hbm.at[gid.at[ch * PIECES + p]],
                rows.at[pl.ds(p * IDX_W, IDX_W)], sem))
        for w in waits:
            w.wait()
        pltpu.sync_copy(rows, out_hbm.at[pl.ds(r0 + ch * CHUNK, CHUNK)])


@jax.jit
def _run(tab, inp2):
    mesh = plsc.VectorSubcoreMesh(core_axis_name="c", subcore_axis_name="s")
    f = pl.kernel(
        _body,
        out_type=jax.ShapeDtypeStruct((B * L, D), jnp.float32),
        mesh=mesh,
        compiler_params=pltpu.CompilerParams(use_tc_tiling_on_sc=False),
        scratch_types=[
            pltpu.VMEM((ROWS_PER_W,), jnp.int32),
            pltpu.VMEM((IDX_ROWS, IDX_W), jnp.int32),
            pltpu.VMEM((CHUNK, D), jnp.float32),
            pltpu.SemaphoreType.DMA,
        ],
    )
    return f(tab, inp2)


def kernel(input, weight):
    tab = weight.reshape(B * V, D)
    inp2 = input.reshape(B * L).astype(jnp.int32)
    out = _run(tab, inp2)
    return out.reshape(B, L, D)


# speedup vs baseline: 1.1293x; 1.0010x over previous
import functools

import jax
import jax.numpy as jnp
from jax import lax
from jax.experimental import pallas as pl
from jax.experimental.pallas import tpu as pltpu
from jax.experimental.pallas import tpu_sc as plsc

B, V, D, L = 1024, 1000, 32, 200
NW = 32
ROWS_PER_W = B * L // NW
BATCH_PER_W = B // NW
IDX_W = 128
IDX_ROWS = ROWS_PER_W // IDX_W
CHUNK = 1280
N_CHUNK = ROWS_PER_W // CHUNK
PIECES = CHUNK // IDX_W
DIV_M, DIV_S = 5243, 20


def _body(tab_hbm, inp_hbm, out_hbm, idx_raw, gid, rows0, rows1,
          semg0, semg1, semw0, semw1):
    rows = (rows0, rows1)
    semg = (semg0, semg1)
    semw = (semw0, semw1)
    c = lax.axis_index("c")
    s = lax.axis_index("s")
    wid = s * 2 + c
    r0 = wid * ROWS_PER_W

    pltpu.sync_copy(inp_hbm.at[pl.ds(r0, ROWS_PER_W)], idx_raw)

    wbase = wid * BATCH_PER_W * V
    lanes = lax.iota(jnp.int32, 16)

    def vec_body(i, carry):
        row = i >> 3
        col = (i & 7) * 16
        pos = i * 16 + lanes
        b_local = lax.shift_right_logical(pos * DIV_M, DIV_S)
        raw = idx_raw[pl.ds(i * 16, 16)]
        gid[row, pl.ds(col, 16)] = raw + (wbase + b_local * V)
        return carry

    lax.fori_loop(0, IDX_ROWS * 8, vec_body, 0)

    def fire(ch):
        b = ch % 2
        return [pltpu.async_copy(
            tab_hbm.at[gid.at[ch * PIECES + p]],
            rows[b].at[pl.ds(p * IDX_W, IDX_W)], semg[b])
            for p in range(PIECES)]

    gw = {0: fire(0)}
    wb = {}
    for ch in range(N_CHUNK):
        b = ch % 2
        for w in gw.pop(ch):
            w.wait()
        if ch + 1 < N_CHUNK:
            if ch - 1 >= 0:
                wb.pop(ch - 1).wait()
            gw[ch + 1] = fire(ch + 1)
        wb[ch] = pltpu.async_copy(
            rows[b], out_hbm.at[pl.ds(r0 + ch * CHUNK, CHUNK)], semw[b])
    for w in wb.values():
        w.wait()


@jax.jit
def _run(tab, inp2):
    mesh = plsc.VectorSubcoreMesh(core_axis_name="c", subcore_axis_name="s")
    f = pl.kernel(
        _body,
        out_type=jax.ShapeDtypeStruct((B * L, D), jnp.float32),
        mesh=mesh,
        compiler_params=pltpu.CompilerParams(use_tc_tiling_on_sc=False),
        scratch_types=[
            pltpu.VMEM((ROWS_PER_W,), jnp.int32),
            pltpu.VMEM((IDX_ROWS, IDX_W), jnp.int32),
            pltpu.VMEM((CHUNK, D), jnp.float32),
            pltpu.VMEM((CHUNK, D), jnp.float32),
            pltpu.SemaphoreType.DMA,
            pltpu.SemaphoreType.DMA,
            pltpu.SemaphoreType.DMA,
            pltpu.SemaphoreType.DMA,
        ],
    )
    return f(tab, inp2)


def kernel(input, weight):
    tab = weight.reshape(B * V, D)
    inp2 = input.reshape(B * L).astype(jnp.int32)
    out = _run(tab, inp2)
    return out.reshape(B, L, D)
